# Initial kernel scaffold; baseline (speedup 1.0000x reference)
import jax
import jax.numpy as jnp
from jax.experimental import pallas as pl


def kernel(atom_list, bond_list, atom_degree_list, bond_degree_list, atom_mask, params):
    raise NotImplementedError("write your pallas kernel here")


# speedup vs baseline: 25.4366x; 25.4366x over previous
import jax
import jax.numpy as jnp
from jax.experimental import pallas as pl

B, L, D, NB = 1024, 64, 8, 128
F, BD, FP, OUT = 128, 16, 32, 1
H = FP * 4
NL = 4
MB = 8


def _body(atom_ref, bond_ref, adi_ref, bdi_ref, mask_ref,
          w_in_ref, af_b1_ref, af_w2_ref, af_b2_ref,
          w1b_ref, nf_b1_ref, nf_w2_ref, nf_b2_ref,
          gw1_ref, gb1_ref, gw2_ref, gb2_ref, gw3_ref, gb3_ref,
          ow1_ref, ob1_ref, ow2_ref, ob2_ref, ow3r_ref, ob3_ref,
          af_out_ref, pred_out_ref):
    fm = lambda a, b: jax.lax.dot_general(
        a, b, (((1,), (0,)), ((), ())), preferred_element_type=jnp.float32)
    relu = lambda x: jnp.maximum(x, 0.0)

    x = atom_ref[...].reshape(MB * L, F)
    t = fm(x, w_in_ref[...])
    af = fm(relu(t[:, :H] + af_b1_ref[...]), af_w2_ref[...]) + af_b2_ref[...]
    pa = t[:, H:]
    nf_b1 = nf_b1_ref[...]
    w1b = w1b_ref[...]

    s_parts = []
    for m in range(MB):
        pa_m = pa[m * L:(m + 1) * L, :]
        bond_m = bond_ref[m]
        ia = adi_ref[m]
        ib = bdi_ref[m]
        oa = (ia == jax.lax.broadcasted_iota(jnp.int32, (D * L, L), 1)
              ).astype(jnp.float32)
        ob = (ib == jax.lax.broadcasted_iota(jnp.int32, (D * L, NB), 1)
              ).astype(jnp.float32)
        ga = fm(oa, pa_m)
        gb = fm(fm(ob, bond_m), w1b)
        h = relu(ga + gb + nf_b1)
        sm = h[0:L]
        for d in range(1, D):
            sm = sm + h[d * L:(d + 1) * L]
        s_parts.append(sm)
    s = jnp.concatenate(s_parts, axis=0)

    nf = relu(fm(s, nf_w2_ref[...]) + float(D) * nf_b2_ref[...])
    for li in range(NL):
        h1 = relu(fm(nf, gw1_ref[li]) + gb1_ref[li])
        h2 = relu(fm(h1, gw2_ref[li]) + gb2_ref[li])
        nf = fm(h2, gw3_ref[li]) + gb3_ref[li]
        af = af + nf
    af_out_ref[...] = af.reshape(MB, L, FP)

    am = relu(af) * mask_ref[...].reshape(MB * L, 1)
    mol = jnp.concatenate(
        [jnp.sum(am[m * L:(m + 1) * L], axis=0, keepdims=True)
         for m in range(MB)], axis=0)
    h1 = relu(fm(mol, ow1_ref[...]) + ob1_ref[...])
    h2 = relu(fm(h1, ow2_ref[...]) + ob2_ref[...])
    pred_out_ref[...] = (jnp.sum(h2 * ow3r_ref[...], axis=1, keepdims=True)
                         + ob3_ref[...])


def kernel(atom_list, bond_list, atom_degree_list, bond_degree_list,
           atom_mask, params):
    p = params
    ad = jnp.clip(atom_degree_list, 0, L - 1).astype(jnp.int32)
    bd = jnp.clip(bond_degree_list, 0, NB - 1).astype(jnp.int32)
    adi = jnp.transpose(ad, (0, 2, 1)).reshape(B, D * L, 1)
    bdi = jnp.transpose(bd, (0, 2, 1)).reshape(B, D * L, 1)

    w_in = jnp.concatenate([p['af_w1'], p['nf_w1'][:F]], axis=1)
    w1b = p['nf_w1'][F:]
    row = lambda v: v.reshape(1, -1)
    gw1 = jnp.stack([p[f'gcn{li}_w1'] for li in range(NL)])
    gb1 = jnp.stack([row(p[f'gcn{li}_b1']) for li in range(NL)])
    gw2 = jnp.stack([p[f'gcn{li}_w2'] for li in range(NL)])
    gb2 = jnp.stack([row(p[f'gcn{li}_b2']) for li in range(NL)])
    gw3 = jnp.stack([p[f'gcn{li}_w3'] for li in range(NL)])
    gb3 = jnp.stack([row(p[f'gcn{li}_b3']) for li in range(NL)])
    mask3 = atom_mask.reshape(B, L, 1)

    full = lambda shape: pl.BlockSpec(shape, lambda i: (0,) * len(shape))
    blk = lambda shape: pl.BlockSpec(shape, lambda i: (i,) + (0,) * (len(shape) - 1))

    in_specs = [
        blk((MB, L, F)), blk((MB, NB, BD)), blk((MB, D * L, 1)),
        blk((MB, D * L, 1)), blk((MB, L, 1)),
        full((F, 2 * H)), full((1, H)), full((H, FP)), full((1, FP)),
        full((BD, H)), full((1, H)), full((H, FP)), full((1, FP)),
        full((NL, FP, H)), full((NL, 1, H)), full((NL, H, H)),
        full((NL, 1, H)), full((NL, H, FP)), full((NL, 1, FP)),
        full((FP, H)), full((1, H)), full((H, H)), full((1, H)),
        full((1, H)), full((1, 1)),
    ]
    out_specs = [blk((MB, L, FP)), blk((MB, 1))]
    out_shape = [jax.ShapeDtypeStruct((B, L, FP), jnp.float32),
                 jax.ShapeDtypeStruct((B, OUT), jnp.float32)]

    af, pred = pl.pallas_call(
        _body, grid=(B // MB,), in_specs=in_specs, out_specs=out_specs,
        out_shape=out_shape,
    )(atom_list, bond_list, adi, bdi, mask3,
      w_in, row(p['af_b1']), p['af_w2'], row(p['af_b2']),
      w1b, row(p['nf_b1']), p['nf_w2'], row(p['nf_b2']),
      gw1, gb1, gw2, gb2, gw3, gb3,
      p['out_w1'], row(p['out_b1']), p['out_w2'], row(p['out_b2']),
      p['out_w3'].reshape(1, H), p['out_b3'].reshape(1, 1))
    return af, pred


# speedup vs baseline: 27.3998x; 1.0772x over previous
import jax
import jax.numpy as jnp
from jax.experimental import pallas as pl

B, L, D, NB = 1024, 64, 8, 128
F, BD, FP, OUT = 128, 16, 32, 1
H = FP * 4
NL = 4
MB = 8
T = L + NB


def _body(atom_ref, bond_ref, adi_ref, bdi_ref, mask_ref,
          w_in_ref, af_b1_ref, af_w2_ref, af_b2_ref,
          w1b_ref, nf_b1_ref, nf_w2_ref, nf_b2_ref,
          g1w_ref, g1b_ref, gw2_ref, gb2_ref, gP_ref, gb31_ref,
          gb3_ref, g3w_ref,
          ow1_ref, ob1_ref, ow2_ref, ob2_ref, ow3r_ref, ob3_ref,
          af_out_ref, pred_out_ref):
    fm = lambda a, b: jax.lax.dot_general(
        a, b, (((1,), (0,)), ((), ())), preferred_element_type=jnp.float32)
    relu = lambda x: jnp.maximum(x, 0.0)

    x = atom_ref[...].reshape(MB * L, F)
    t = fm(x, w_in_ref[...])
    af = fm(relu(t[:, :H] + af_b1_ref[...]), af_w2_ref[...]) + af_b2_ref[...]
    pa = t[:, H:]
    pb = fm(bond_ref[...].reshape(MB * NB, BD), w1b_ref[...])
    nf_b1 = nf_b1_ref[...]

    s_parts = []
    for m in range(MB):
        tb = jnp.concatenate(
            [pa[m * L:(m + 1) * L], pb[m * NB:(m + 1) * NB]], axis=0)
        ia = adi_ref[m]
        ib = bdi_ref[m]
        ii = jax.lax.broadcasted_iota(jnp.int32, (D * L, T), 1)
        oc = ((ia == ii) | (ib == ii)).astype(jnp.float32)
        h = relu(fm(oc, tb) + nf_b1)
        sm = h[0:L]
        for d in range(1, D):
            sm = sm + h[d * L:(d + 1) * L]
        s_parts.append(sm)
    s = jnp.concatenate(s_parts, axis=0)

    nf = relu(fm(s, nf_w2_ref[...]) + float(D) * nf_b2_ref[...])
    h1 = relu(fm(nf, g1w_ref[...]) + g1b_ref[...])
    for li in range(NL - 1):
        h2 = relu(fm(h1, gw2_ref[li]) + gb2_ref[li])
        z = fm(h2, gP_ref[li])
        af = af + z[:, H:] + gb3_ref[li]
        h1 = relu(z[:, :H] + gb31_ref[li])
    h2 = relu(fm(h1, gw2_ref[NL - 1]) + gb2_ref[NL - 1])
    af = af + fm(h2, g3w_ref[...]) + gb3_ref[NL - 1]
    af_out_ref[...] = af.reshape(MB, L, FP)

    am = relu(af) * mask_ref[...].reshape(MB * L, 1)
    mol = jnp.concatenate(
        [jnp.sum(am[m * L:(m + 1) * L], axis=0, keepdims=True)
         for m in range(MB)], axis=0)
    h1 = relu(fm(mol, ow1_ref[...]) + ob1_ref[...])
    h2 = relu(fm(h1, ow2_ref[...]) + ob2_ref[...])
    pred_out_ref[...] = (jnp.sum(h2 * ow3r_ref[...], axis=1, keepdims=True)
                         + ob3_ref[...])


def kernel(atom_list, bond_list, atom_degree_list, bond_degree_list,
           atom_mask, params):
    p = params
    ad = jnp.clip(atom_degree_list, 0, L - 1).astype(jnp.int32)
    bd = jnp.clip(bond_degree_list, 0, NB - 1).astype(jnp.int32)
    adi = jnp.transpose(ad, (0, 2, 1)).reshape(B, D * L, 1)
    bdi = jnp.transpose(bd, (0, 2, 1)).reshape(B, D * L, 1) + L

    w_in = jnp.concatenate([p['af_w1'], p['nf_w1'][:F]], axis=1)
    w1b = p['nf_w1'][F:]
    row = lambda v: v.reshape(1, -1)
    gP = jnp.stack([
        jnp.concatenate([p[f'gcn{li}_w3'] @ p[f'gcn{li + 1}_w1'],
                         p[f'gcn{li}_w3']], axis=1)
        for li in range(NL - 1)])
    gb31 = jnp.stack([
        row(p[f'gcn{li}_b3'] @ p[f'gcn{li + 1}_w1'] + p[f'gcn{li + 1}_b1'])
        for li in range(NL - 1)])
    gw2 = jnp.stack([p[f'gcn{li}_w2'] for li in range(NL)])
    gb2 = jnp.stack([row(p[f'gcn{li}_b2']) for li in range(NL)])
    gb3 = jnp.stack([row(p[f'gcn{li}_b3']) for li in range(NL)])
    mask3 = atom_mask.reshape(B, L, 1)

    full = lambda shape: pl.BlockSpec(shape, lambda i: (0,) * len(shape))
    blk = lambda shape: pl.BlockSpec(shape, lambda i: (i,) + (0,) * (len(shape) - 1))

    in_specs = [
        blk((MB, L, F)), blk((MB, NB, BD)), blk((MB, D * L, 1)),
        blk((MB, D * L, 1)), blk((MB, L, 1)),
        full((F, 2 * H)), full((1, H)), full((H, FP)), full((1, FP)),
        full((BD, H)), full((1, H)), full((H, FP)), full((1, FP)),
        full((FP, H)), full((1, H)), full((NL, H, H)), full((NL, 1, H)),
        full((NL - 1, H, H + FP)), full((NL - 1, 1, H)),
        full((NL, 1, FP)), full((H, FP)),
        full((FP, H)), full((1, H)), full((H, H)), full((1, H)),
        full((1, H)), full((1, 1)),
    ]
    out_specs = [blk((MB, L, FP)), blk((MB, 1))]
    out_shape = [jax.ShapeDtypeStruct((B, L, FP), jnp.float32),
                 jax.ShapeDtypeStruct((B, OUT), jnp.float32)]

    af, pred = pl.pallas_call(
        _body, grid=(B // MB,), in_specs=in_specs, out_specs=out_specs,
        out_shape=out_shape,
    )(atom_list, bond_list, adi, bdi, mask3,
      w_in, row(p['af_b1']), p['af_w2'], row(p['af_b2']),
      w1b, row(p['nf_b1']), p['nf_w2'], row(p['nf_b2']),
      p['gcn0_w1'], row(p['gcn0_b1']), gw2, gb2, gP, gb31,
      gb3, p[f'gcn{NL - 1}_w3'],
      p['out_w1'], row(p['out_b1']), p['out_w2'], row(p['out_b2']),
      p['out_w3'].reshape(1, H), p['out_b3'].reshape(1, 1))
    return af, pred


# speedup vs baseline: 41.6696x; 1.5208x over previous
import jax
import jax.numpy as jnp
from jax.experimental import pallas as pl

B, L, D, NB = 1024, 64, 8, 128
F, BD, FP, OUT = 128, 16, 32, 1
H = FP * 4
NL = 4
MB = 16
T = L + NB


def _body(atom_ref, bond_ref, adi_ref, bdi_ref, mmask_ref,
          w_in_ref, af_b1_ref, af_w2_ref, af_b2_ref,
          w1b_ref, nf_b1_ref, nf_w2_ref, nf_b2_ref,
          g1w_ref, g1b_ref, gw2_ref, gb2_ref, gP_ref, gb31_ref,
          gb3_ref, g3w_ref,
          ow1_ref, ob1_ref, ow2_ref, ob2_ref, ow3r_ref, ob3_ref,
          af_out_ref, pred_out_ref, s_ref):
    bf = jnp.bfloat16
    fm = lambda a, b: jax.lax.dot_general(
        a, b, (((1,), (0,)), ((), ())), preferred_element_type=jnp.float32)
    relu = lambda x: jnp.maximum(x, 0.0)

    x = atom_ref[...].reshape(MB * L, F)
    t = fm(x, w_in_ref[...])
    af = fm(relu(t[:, :H] + af_b1_ref[...]), af_w2_ref[...]) + af_b2_ref[...]
    pa = t[:, H:].astype(bf)
    pb = fm(bond_ref[...].reshape(MB * NB, BD),
            w1b_ref[...]).astype(bf)
    nf_b1 = nf_b1_ref[...]

    ii = jax.lax.broadcasted_iota(jnp.int32, (L, T), 1)
    for m in range(MB):
        tb = jnp.concatenate(
            [pa[m * L:(m + 1) * L], pb[m * NB:(m + 1) * NB]], axis=0)
        ia = jnp.clip(adi_ref[m], 0, L - 1)
        ib = jnp.clip(bdi_ref[m], 0, NB - 1) + L
        sm = None
        for d in range(D):
            oc = ((ia[:, d:d + 1] == ii) | (ib[:, d:d + 1] == ii)
                  ).astype(bf)
            hd = relu(fm(oc, tb) + nf_b1)
            sm = hd if sm is None else sm + hd
        s_ref[m * L:(m + 1) * L, :] = sm
    s = s_ref[...]

    nf = relu(fm(s, nf_w2_ref[...]) + float(D) * nf_b2_ref[...])
    h1 = relu(fm(nf, g1w_ref[...]) + g1b_ref[...])
    for li in range(NL - 1):
        h2 = relu(fm(h1, gw2_ref[li]) + gb2_ref[li])
        z = fm(h2, gP_ref[li])
        af = af + z[:, H:] + gb3_ref[li]
        h1 = relu(z[:, :H] + gb31_ref[li])
    h2 = relu(fm(h1, gw2_ref[NL - 1]) + gb2_ref[NL - 1])
    af = af + fm(h2, g3w_ref[...]) + gb3_ref[NL - 1]
    af_out_ref[...] = af.reshape(MB, L, FP)

    mol = fm(mmask_ref[0], relu(af))
    h1 = relu(fm(mol, ow1_ref[...]) + ob1_ref[...])
    h2 = relu(fm(h1, ow2_ref[...]) + ob2_ref[...])
    pred_out_ref[...] = (jnp.sum(h2 * ow3r_ref[...], axis=1, keepdims=True)
                         + ob3_ref[...])


def kernel(atom_list, bond_list, atom_degree_list, bond_degree_list,
           atom_mask, params):
    p = params
    mmask = (atom_mask.reshape(B // MB, MB, 1, L)
             * jnp.eye(MB)[None, :, :, None]).reshape(B // MB, MB, MB * L)

    w_in = jnp.concatenate([p['af_w1'], p['nf_w1'][:F]], axis=1)
    w1b = p['nf_w1'][F:]
    row = lambda v: v.reshape(1, -1)
    gP = jnp.stack([
        jnp.concatenate([p[f'gcn{li}_w3'] @ p[f'gcn{li + 1}_w1'],
                         p[f'gcn{li}_w3']], axis=1)
        for li in range(NL - 1)])
    gb31 = jnp.stack([
        row(p[f'gcn{li}_b3'] @ p[f'gcn{li + 1}_w1'] + p[f'gcn{li + 1}_b1'])
        for li in range(NL - 1)])
    gw2 = jnp.stack([p[f'gcn{li}_w2'] for li in range(NL)])
    gb2 = jnp.stack([row(p[f'gcn{li}_b2']) for li in range(NL)])
    gb3 = jnp.stack([row(p[f'gcn{li}_b3']) for li in range(NL)])

    full = lambda shape: pl.BlockSpec(shape, lambda i: (0,) * len(shape))
    blk = lambda shape: pl.BlockSpec(shape, lambda i: (i,) + (0,) * (len(shape) - 1))

    in_specs = [
        blk((MB, L, F)), blk((MB, NB, BD)), blk((MB, L, D)),
        blk((MB, L, D)), blk((1, MB, MB * L)),
        full((F, 2 * H)), full((1, H)), full((H, FP)), full((1, FP)),
        full((BD, H)), full((1, H)), full((H, FP)), full((1, FP)),
        full((FP, H)), full((1, H)), full((NL, H, H)), full((NL, 1, H)),
        full((NL - 1, H, H + FP)), full((NL - 1, 1, H)),
        full((NL, 1, FP)), full((H, FP)),
        full((FP, H)), full((1, H)), full((H, H)), full((1, H)),
        full((1, H)), full((1, 1)),
    ]
    out_specs = [blk((MB, L, FP)), blk((MB, 1))]
    out_shape = [jax.ShapeDtypeStruct((B, L, FP), jnp.float32),
                 jax.ShapeDtypeStruct((B, OUT), jnp.float32)]

    from jax.experimental.pallas import tpu as pltpu
    af, pred = pl.pallas_call(
        _body, grid=(B // MB,), in_specs=in_specs, out_specs=out_specs,
        out_shape=out_shape,
        scratch_shapes=[pltpu.VMEM((MB * L, H), jnp.float32)],
    )(atom_list, bond_list, atom_degree_list, bond_degree_list, mmask,
      w_in, row(p['af_b1']), p['af_w2'], row(p['af_b2']),
      w1b, row(p['nf_b1']), p['nf_w2'], row(p['nf_b2']),
      p['gcn0_w1'], row(p['gcn0_b1']), gw2, gb2, gP, gb31,
      gb3, p[f'gcn{NL - 1}_w3'],
      p['out_w1'], row(p['out_b1']), p['out_w2'], row(p['out_b2']),
      p['out_w3'].reshape(1, H), p['out_b3'].reshape(1, 1))
    return af, pred
